# baseline (speedup 1.0000x reference)
import functools

import jax
import jax.numpy as jnp
from jax import lax
from jax.experimental import pallas as pl
from jax.experimental.pallas import tpu as pltpu

S = 2048
D = 1024
DFF = 4096
E = 8
TOPK = 2
P = S * TOPK
TM = 128
NT = P // TM + E
MP = NT * TM

_F32 = jnp.float32


def _fiota(shape, dim):
    return lax.broadcasted_iota(jnp.int32, shape, dim).astype(_F32)


def _router_kernel(x_ref, wr_ref, br_ref, pos_ref, pp_ref, te_ref, tv_ref):
    x = x_ref[...]
    logits = lax.dot_general(x, wr_ref[...], (((1,), (0,)), ((), ())),
                             preferred_element_type=_F32) + br_ref[...]
    iota_e = _fiota((S, E), 1)
    m0 = jnp.max(logits, axis=1, keepdims=True)
    e0 = jnp.min(jnp.where(logits == m0, iota_e, float(E)), axis=1,
                 keepdims=True)
    masked = jnp.where(iota_e == e0, -1e30, logits)
    m1 = jnp.max(masked, axis=1, keepdims=True)
    e1 = jnp.min(jnp.where(masked == m1, iota_e, float(E)), axis=1,
                 keepdims=True)
    t = jnp.exp(m1 - m0)
    p0 = 1.0 / (1.0 + t)
    p1 = t / (1.0 + t)

    e_pair = jnp.concatenate([e0, e1], axis=0)
    p_pair = jnp.concatenate([p0, p1], axis=0)
    iota_pe = _fiota((P, E), 1)
    onehot = (e_pair == iota_pe).astype(_F32)

    chunk = 512
    ranks = []
    for c in range(P // chunk):
        row_g = _fiota((chunk, P), 0) + float(c * chunk)
        col = _fiota((chunk, P), 1)
        tri = (col < row_g).astype(_F32)
        ranks.append(lax.dot_general(tri, onehot, (((1,), (0,)), ((), ())),
                                     preferred_element_type=_F32))
    rank = jnp.concatenate(ranks, axis=0)

    counts = jnp.sum(onehot, axis=0, keepdims=True)
    ntiles = jnp.floor((counts + float(TM - 1)) / float(TM))
    aligned = ntiles * float(TM)
    ir = _fiota((E, E), 0)
    ic = _fiota((E, E), 1)
    up = (ir < ic).astype(_F32)
    start = lax.dot_general(aligned, up, (((1,), (0,)), ((), ())),
                            preferred_element_type=_F32)

    pos = (jnp.sum(onehot * start, axis=1, keepdims=True)
           + jnp.sum(onehot * rank, axis=1, keepdims=True))
    pos_ref[...] = pos.astype(jnp.int32)
    pp_ref[...] = p_pair

    ts = start / float(TM)
    tend = ts + ntiles
    tt = _fiota((NT, E), 0)
    within = jnp.logical_and(tt >= ts, tt < tend).astype(_F32)
    iota_te = _fiota((NT, E), 1)
    texp = jnp.sum(within * iota_te, axis=1, keepdims=True)
    tval = jnp.sum(within, axis=1, keepdims=True)
    iota_1e = _fiota((1, E), 1)
    last_e = jnp.max(jnp.where(counts > 0.0, iota_1e, -1.0))
    texp = jnp.where(tval > 0.0, texp, last_e)
    te_ref[...] = texp.astype(jnp.int32)
    tv_ref[...] = tval.astype(jnp.int32)


def _route(x, wr, br):
    return pl.pallas_call(
        _router_kernel,
        out_shape=[
            jax.ShapeDtypeStruct((P, 1), jnp.int32),
            jax.ShapeDtypeStruct((P, 1), _F32),
            jax.ShapeDtypeStruct((NT, 1), jnp.int32),
            jax.ShapeDtypeStruct((NT, 1), jnp.int32),
        ],
    )(x, wr, br)


def _dispatch_kernel(pos_ref, x_ref, xg_ref):
    UNROLL = 8
    def body(u, _):
        for j in range(UNROLL):
            i = u * UNROLL + j
            tok = lax.rem(i, S)
            r = pos_ref[i]
            xg_ref[pl.ds(r, 1), :] = x_ref[pl.ds(tok, 1), :]
        return 0
    lax.fori_loop(0, P // UNROLL, body, 0)


def _dispatch(pos, x3):
    return pl.pallas_call(
        _dispatch_kernel,
        in_specs=[
            pl.BlockSpec(memory_space=pltpu.SMEM),
            pl.BlockSpec(memory_space=pltpu.VMEM),
        ],
        out_specs=pl.BlockSpec(memory_space=pltpu.VMEM),
        out_shape=jax.ShapeDtypeStruct((MP, D), _F32),
    )(pos, x3)


def _mm1_kernel(te_ref, tv_ref, x_ref, w_ref, b_ref, h_ref):
    t = pl.program_id(0)

    @pl.when(tv_ref[t] > 0)
    def _():
        acc = lax.dot_general(x_ref[...].astype(jnp.bfloat16), w_ref[0],
                              (((1,), (0,)), ((), ())),
                              preferred_element_type=_F32) + b_ref[0]
        h = 0.5 * acc * (1.0 + lax.erf(acc * 0.7071067811865476))
        h_ref[...] = h.astype(jnp.bfloat16)


def _mm2_kernel(te_ref, tv_ref, h_ref, w_ref, b_ref, y_ref):
    t = pl.program_id(0)

    @pl.when(tv_ref[t] > 0)
    def _():
        y_ref[...] = lax.dot_general(h_ref[...], w_ref[0],
                                     (((1,), (0,)), ((), ())),
                                     preferred_element_type=_F32) + b_ref[0]


def _mm1(te, tv, xg, w1, b1):
    grid_spec = pltpu.PrefetchScalarGridSpec(
        num_scalar_prefetch=2,
        grid=(NT,),
        in_specs=[
            pl.BlockSpec((TM, D), lambda t, te, tv: (t, 0)),
            pl.BlockSpec((1, D, DFF), lambda t, te, tv: (te[t], 0, 0)),
            pl.BlockSpec((1, 1, DFF), lambda t, te, tv: (te[t], 0, 0)),
        ],
        out_specs=pl.BlockSpec((TM, DFF), lambda t, te, tv: (t, 0)),
    )
    return pl.pallas_call(
        _mm1_kernel,
        grid_spec=grid_spec,
        out_shape=jax.ShapeDtypeStruct((MP, DFF), jnp.bfloat16),
    )(te[:, 0], tv[:, 0], xg, w1, b1.reshape(E, 1, DFF))


def _mm2(te, tv, h, w2, b2):
    grid_spec = pltpu.PrefetchScalarGridSpec(
        num_scalar_prefetch=2,
        grid=(NT,),
        in_specs=[
            pl.BlockSpec((TM, DFF), lambda t, te, tv: (t, 0)),
            pl.BlockSpec((1, DFF, D), lambda t, te, tv: (te[t], 0, 0)),
            pl.BlockSpec((1, 1, D), lambda t, te, tv: (te[t], 0, 0)),
        ],
        out_specs=pl.BlockSpec((TM, D), lambda t, te, tv: (t, 0)),
    )
    return pl.pallas_call(
        _mm2_kernel,
        grid_spec=grid_spec,
        out_shape=jax.ShapeDtypeStruct((MP, D), _F32),
    )(te[:, 0], tv[:, 0], h, w2, b2.reshape(E, 1, D))


def _combine_kernel(pos_ref, pp_ref, y_ref, o_ref):
    UNROLL = 8
    def body(u, _):
        for j in range(UNROLL):
            tok = u * UNROLL + j
            r0 = pos_ref[tok]
            r1 = pos_ref[tok + S]
            o_ref[pl.ds(tok, 1), :] = (pp_ref[tok] * y_ref[pl.ds(r0, 1), :]
                                       + pp_ref[tok + S] * y_ref[pl.ds(r1, 1), :])
        return 0
    lax.fori_loop(0, S // UNROLL, body, 0)


def _combine(pos, pp, y3):
    return pl.pallas_call(
        _combine_kernel,
        in_specs=[
            pl.BlockSpec(memory_space=pltpu.SMEM),
            pl.BlockSpec(memory_space=pltpu.SMEM),
            pl.BlockSpec(memory_space=pltpu.VMEM),
        ],
        out_specs=pl.BlockSpec(memory_space=pltpu.VMEM),
        out_shape=jax.ShapeDtypeStruct((S, D), _F32),
    )(pos, pp, y3)


@jax.jit
def kernel(hidden_states, W1, b1, W2, b2, Wr, br):
    x = hidden_states.reshape(S, D)
    pos, pp, te, tv = _route(x, Wr, br.reshape(1, E))
    xg = _dispatch(pos.reshape(P), x)
    h = _mm1(te, tv, xg, W1.astype(jnp.bfloat16), b1)
    y = _mm2(te, tv, h, W2.astype(jnp.bfloat16), b2)
    out = _combine(pos.reshape(P), pp.reshape(P), y)
    return out.reshape(1, S, D)


# speedup vs baseline: 1.2578x; 1.2578x over previous
import functools

import jax
import jax.numpy as jnp
from jax import lax
from jax.experimental import pallas as pl
from jax.experimental.pallas import tpu as pltpu
from jax.experimental.pallas import tpu_sc as plsc

S = 2048
D = 1024
DFF = 4096
E = 8
TOPK = 2
P = S * TOPK
TM = 128
NT = P // TM + E
MP = NT * TM

_F32 = jnp.float32


def _fiota(shape, dim):
    return lax.broadcasted_iota(jnp.int32, shape, dim).astype(_F32)


def _router_kernel(x_ref, wr_ref, br_ref, pos_ref, pp_ref, te_ref, tv_ref):
    x = x_ref[...]
    logits = lax.dot_general(x, wr_ref[...], (((1,), (0,)), ((), ())),
                             preferred_element_type=_F32) + br_ref[...]
    iota_e = _fiota((S, E), 1)
    m0 = jnp.max(logits, axis=1, keepdims=True)
    e0 = jnp.min(jnp.where(logits == m0, iota_e, float(E)), axis=1,
                 keepdims=True)
    masked = jnp.where(iota_e == e0, -1e30, logits)
    m1 = jnp.max(masked, axis=1, keepdims=True)
    e1 = jnp.min(jnp.where(masked == m1, iota_e, float(E)), axis=1,
                 keepdims=True)
    t = jnp.exp(m1 - m0)
    p0 = 1.0 / (1.0 + t)
    p1 = t / (1.0 + t)

    e_pair = jnp.concatenate([e0, e1], axis=0)
    p_pair = jnp.concatenate([p0, p1], axis=0)
    iota_pe = _fiota((P, E), 1)
    onehot = (e_pair == iota_pe).astype(_F32)

    chunk = 512
    ranks = []
    for c in range(P // chunk):
        row_g = _fiota((chunk, P), 0) + float(c * chunk)
        col = _fiota((chunk, P), 1)
        tri = (col < row_g).astype(_F32)
        ranks.append(lax.dot_general(tri, onehot, (((1,), (0,)), ((), ())),
                                     preferred_element_type=_F32))
    rank = jnp.concatenate(ranks, axis=0)

    counts = jnp.sum(onehot, axis=0, keepdims=True)
    ntiles = jnp.floor((counts + float(TM - 1)) / float(TM))
    aligned = ntiles * float(TM)
    ir = _fiota((E, E), 0)
    ic = _fiota((E, E), 1)
    up = (ir < ic).astype(_F32)
    start = lax.dot_general(aligned, up, (((1,), (0,)), ((), ())),
                            preferred_element_type=_F32)

    pos = (jnp.sum(onehot * start, axis=1, keepdims=True)
           + jnp.sum(onehot * rank, axis=1, keepdims=True))
    pos_ref[...] = pos.astype(jnp.int32)
    pp_ref[...] = p_pair

    ts = start / float(TM)
    tend = ts + ntiles
    tt = _fiota((NT, E), 0)
    within = jnp.logical_and(tt >= ts, tt < tend).astype(_F32)
    iota_te = _fiota((NT, E), 1)
    texp = jnp.sum(within * iota_te, axis=1, keepdims=True)
    tval = jnp.sum(within, axis=1, keepdims=True)
    iota_1e = _fiota((1, E), 1)
    last_e = jnp.max(jnp.where(counts > 0.0, iota_1e, -1.0))
    texp = jnp.where(tval > 0.0, texp, last_e)
    te_ref[...] = texp.astype(jnp.int32)
    tv_ref[...] = tval.astype(jnp.int32)


def _route(x, wr, br):
    return pl.pallas_call(
        _router_kernel,
        out_shape=[
            jax.ShapeDtypeStruct((P, 1), jnp.int32),
            jax.ShapeDtypeStruct((P, 1), _F32),
            jax.ShapeDtypeStruct((NT, 1), jnp.int32),
            jax.ShapeDtypeStruct((NT, 1), jnp.int32),
        ],
    )(x, wr, br)


_SC_NC = 2
_SC_NS = 16
_SC_NW = _SC_NC * _SC_NS
_SC_CH = 64


def _sc_dispatch_body(x_hbm, pos_hbm, xg_hbm, idx_v, rows_v, sem):
    wid = lax.axis_index("s") * _SC_NC + lax.axis_index("c")
    base = wid * (P // _SC_NW)
    for c in range(P // _SC_NW // _SC_CH):
        b = base + c * _SC_CH
        pltpu.sync_copy(pos_hbm.at[pl.ds(b, _SC_CH)], idx_v)
        pltpu.sync_copy(x_hbm.at[pl.ds(lax.rem(b, S), _SC_CH)], rows_v)
        pltpu.async_copy(rows_v, xg_hbm.at[idx_v], sem).wait()


def _dispatch(pos, x):
    f = functools.partial(
        pl.kernel,
        mesh=plsc.VectorSubcoreMesh(core_axis_name="c", subcore_axis_name="s"),
        out_type=jax.ShapeDtypeStruct((MP, D), _F32),
        scratch_types=[
            pltpu.VMEM((_SC_CH,), jnp.int32),
            pltpu.VMEM((_SC_CH, D), _F32),
            pltpu.SemaphoreType.DMA,
        ],
    )(_sc_dispatch_body)
    return f(x, pos)


def _mm1_kernel(te_ref, tv_ref, x_ref, w_ref, b_ref, h_ref):
    t = pl.program_id(0)

    @pl.when(tv_ref[t] > 0)
    def _():
        acc = lax.dot_general(x_ref[...], w_ref[0], (((1,), (0,)), ((), ())),
                              preferred_element_type=_F32) + b_ref[0]
        h_ref[...] = 0.5 * acc * (1.0 + lax.erf(acc * 0.7071067811865476))


def _mm2_kernel(te_ref, tv_ref, h_ref, w_ref, b_ref, y_ref):
    t = pl.program_id(0)

    @pl.when(tv_ref[t] > 0)
    def _():
        y_ref[...] = lax.dot_general(h_ref[...], w_ref[0],
                                     (((1,), (0,)), ((), ())),
                                     preferred_element_type=_F32) + b_ref[0]


def _mm1(te, tv, xg, w1, b1):
    grid_spec = pltpu.PrefetchScalarGridSpec(
        num_scalar_prefetch=2,
        grid=(NT,),
        in_specs=[
            pl.BlockSpec((TM, D), lambda t, te, tv: (t, 0)),
            pl.BlockSpec((1, D, DFF), lambda t, te, tv: (te[t], 0, 0)),
            pl.BlockSpec((1, 1, DFF), lambda t, te, tv: (te[t], 0, 0)),
        ],
        out_specs=pl.BlockSpec((TM, DFF), lambda t, te, tv: (t, 0)),
    )
    return pl.pallas_call(
        _mm1_kernel,
        grid_spec=grid_spec,
        out_shape=jax.ShapeDtypeStruct((MP, DFF), _F32),
    )(te[:, 0], tv[:, 0], xg, w1, b1.reshape(E, 1, DFF))


def _mm2(te, tv, h, w2, b2):
    grid_spec = pltpu.PrefetchScalarGridSpec(
        num_scalar_prefetch=2,
        grid=(NT,),
        in_specs=[
            pl.BlockSpec((TM, DFF), lambda t, te, tv: (t, 0)),
            pl.BlockSpec((1, DFF, D), lambda t, te, tv: (te[t], 0, 0)),
            pl.BlockSpec((1, 1, D), lambda t, te, tv: (te[t], 0, 0)),
        ],
        out_specs=pl.BlockSpec((TM, D), lambda t, te, tv: (t, 0)),
    )
    return pl.pallas_call(
        _mm2_kernel,
        grid_spec=grid_spec,
        out_shape=jax.ShapeDtypeStruct((MP, D), _F32),
    )(te[:, 0], tv[:, 0], h, w2, b2.reshape(E, 1, D))


def _combine_kernel(pos_ref, pp_ref, y_ref, o_ref):
    UNROLL = 8
    def body(u, _):
        for j in range(UNROLL):
            tok = u * UNROLL + j
            r0 = pos_ref[tok]
            r1 = pos_ref[tok + S]
            o_ref[pl.ds(tok, 1), :] = (pp_ref[tok] * y_ref[pl.ds(r0, 1), :]
                                       + pp_ref[tok + S] * y_ref[pl.ds(r1, 1), :])
        return 0
    lax.fori_loop(0, S // UNROLL, body, 0)


def _combine(pos, pp, y3):
    return pl.pallas_call(
        _combine_kernel,
        in_specs=[
            pl.BlockSpec(memory_space=pltpu.SMEM),
            pl.BlockSpec(memory_space=pltpu.SMEM),
            pl.BlockSpec(memory_space=pltpu.VMEM),
        ],
        out_specs=pl.BlockSpec(memory_space=pltpu.VMEM),
        out_shape=jax.ShapeDtypeStruct((S, D), _F32),
    )(pos, pp, y3)


@jax.jit
def kernel(hidden_states, W1, b1, W2, b2, Wr, br):
    x = hidden_states.reshape(S, D)
    pos, pp, te, tv = _route(x, Wr, br.reshape(1, E))
    xg = _dispatch(pos.reshape(P), x)
    h = _mm1(te, tv, xg, W1, b1)
    y = _mm2(te, tv, h, W2, b2)
    out = _combine(pos.reshape(P), pp.reshape(P), y)
    return out.reshape(1, S, D)
